# baseline (speedup 1.0000x reference)
import functools

import jax
import jax.numpy as jnp
from jax.experimental import pallas as pl

_ROT0 = (13, 15, 26, 6)
_ROT1 = (17, 29, 16, 24)
_ROUND_ROTS = (_ROT0, _ROT1, _ROT0, _ROT1, _ROT0)
_KS = (0, 1, 0x1BD11BDB)


def _rotl(v, d):
    return (v << jnp.uint32(d)) | (v >> jnp.uint32(32 - d))


def _masker_body(inp_ref, mm_ref, rm_ref, out_ref, *, block_rows, ncols):
    i = pl.program_id(0)
    inp = inp_ref[...]
    shape = inp.shape
    mmw = mm_ref[...].astype(jnp.int32)
    rmw = rm_ref[...].astype(jnp.int32)
    base = inp & (mmw - 1)

    row = jax.lax.broadcasted_iota(jnp.int32, shape, 0)
    col = jax.lax.broadcasted_iota(jnp.int32, shape, 1)
    ctr = (i * (block_rows * ncols) + row * ncols + col).astype(jnp.uint32)

    x1 = ctr + jnp.uint32(1)
    x0 = x1
    x1 = x0 ^ _rotl(x1, _ROT0[0])
    for r in _ROT0[1:]:
        x0 = x0 + x1
        x1 = x0 ^ _rotl(x1, r)
    x0 = x0 + jnp.uint32(_KS[1])
    x1 = x1 + jnp.uint32(_KS[2] + 1)
    for g in range(1, 5):
        for r in _ROUND_ROTS[g]:
            x0 = x0 + x1
            x1 = x0 ^ _rotl(x1, r)
        j = g + 1
        if _KS[j % 3]:
            x0 = x0 + jnp.uint32(_KS[j % 3])
        x1 = x1 + jnp.uint32((_KS[(j + 1) % 3] + j) & 0xFFFFFFFF)
    bits = (x0 ^ x1).astype(jnp.int32)

    sign = bits >> 31
    rv_bits = (sign ^ jnp.int32(-1)) & jnp.int32(0x3F800000)
    rsel = rmw - 1
    out_ref[...] = (base & rsel) | (rv_bits & ~rsel)


def kernel(input, mask_mask, rand_mask):
    nrows, ncols = input.shape
    block_rows = 256
    grid = nrows // block_rows
    body = functools.partial(_masker_body, block_rows=block_rows, ncols=ncols)
    spec = pl.BlockSpec((block_rows, ncols), lambda i: (i, 0))
    inp32 = input.view(jnp.int32)
    mm8 = mask_mask.view(jnp.int8)
    rm8 = rand_mask.view(jnp.int8)
    out32 = pl.pallas_call(
        body,
        grid=(grid,),
        in_specs=[spec, spec, spec],
        out_specs=spec,
        out_shape=jax.ShapeDtypeStruct(input.shape, jnp.int32),
    )(inp32, mm8, rm8)
    return out32.view(input.dtype)


# speedup vs baseline: 1.3026x; 1.3026x over previous
import functools

import jax
import jax.numpy as jnp
from jax.experimental import pallas as pl

_ROT0 = (13, 15, 26, 6)
_ROT1 = (17, 29, 16, 24)
_ROUND_ROTS = (_ROT0, _ROT1, _ROT0, _ROT1, _ROT0)
_KS = (0, 1, 0x1BD11BDB)


def _rotl(v, d):
    return (v << jnp.uint32(d)) | (v >> jnp.uint32(32 - d))


def _masker_body(inp_ref, mm_ref, rm_ref, out_ref, *, block_rows, ncols):
    i = pl.program_id(0)
    inp = jax.lax.bitcast_convert_type(inp_ref[...], jnp.int32)
    shape = inp.shape
    mmw = mm_ref[...].astype(jnp.int32)
    rmw = rm_ref[...].astype(jnp.int32)
    base = inp & (mmw - 1)

    row = jax.lax.broadcasted_iota(jnp.int32, shape, 0)
    col = jax.lax.broadcasted_iota(jnp.int32, shape, 1)
    x1 = (i * (block_rows * ncols) + 1 + row * ncols + col).astype(jnp.uint32)

    x0 = x1
    x1 = x0 ^ _rotl(x1, _ROT0[0])
    for r in _ROT0[1:]:
        x0 = x0 + x1
        x1 = x0 ^ _rotl(x1, r)
    x0 = x0 + jnp.uint32(_KS[1])
    x1 = x1 + jnp.uint32(_KS[2] + 1)
    for g in range(1, 5):
        for r in _ROUND_ROTS[g]:
            x0 = x0 + x1
            x1 = x0 ^ _rotl(x1, r)
        j = g + 1
        if _KS[j % 3]:
            x0 = x0 + jnp.uint32(_KS[j % 3])
        x1 = x1 + jnp.uint32((_KS[(j + 1) % 3] + j) & 0xFFFFFFFF)
    bits = (x0 ^ x1).astype(jnp.int32)

    sign = bits >> 31
    rv_bits = (sign ^ jnp.int32(-1)) & jnp.int32(0x3F800000)
    nrsel = -rmw
    out_ref[...] = jax.lax.bitcast_convert_type(
        base ^ ((base ^ rv_bits) & nrsel), jnp.float32)


def kernel(input, mask_mask, rand_mask):
    nrows, ncols = input.shape
    block_rows = 256
    grid = nrows // block_rows
    body = functools.partial(_masker_body, block_rows=block_rows, ncols=ncols)
    spec = pl.BlockSpec((block_rows, ncols), lambda i: (i, 0))
    mm8 = mask_mask.view(jnp.int8)
    rm8 = rand_mask.view(jnp.int8)
    return pl.pallas_call(
        body,
        grid=(grid,),
        in_specs=[spec, spec, spec],
        out_specs=spec,
        out_shape=jax.ShapeDtypeStruct(input.shape, input.dtype),
    )(input, mm8, rm8)
